# Initial kernel scaffold; baseline (speedup 1.0000x reference)
import jax
import jax.numpy as jnp
from jax.experimental import pallas as pl


def kernel(history_data, future_data, batch_seen, epoch, train, edge_index, params):
    raise NotImplementedError("write your pallas kernel here")


# speedup vs baseline: 48.8070x; 48.8070x over previous
import jax
import jax.numpy as jnp
from jax import lax
from jax.experimental import pallas as pl

N = 307
E = 2763
B = 8
T = 12
T_OUT = 12
D = 96
H = 4
HD = D // H
FF = 256
SPD = 288
NL = 3
TN = T * N


def _ln(x, g, b):
    m = jnp.mean(x, axis=-1, keepdims=True)
    v = jnp.mean((x - m) ** 2, axis=-1, keepdims=True)
    return (x - m) / jnp.sqrt(v + 1e-5) * g + b


def _full(shape):
    return pl.BlockSpec(shape, lambda *_: tuple(0 for _ in shape))


def _params(p, names):
    args, specs = [], []
    for nm in names:
        a = p[nm]
        if a.ndim == 1:
            a = a.reshape(1, -1)
        args.append(a)
        specs.append(_full(a.shape))
    return args, specs


def _embed_body(x_ref, win_ref, bin_ref, tod_ref, dow_ref, out_ref):
    x = x_ref[...].reshape(N, 3)
    h1 = (x[:, 0:1] * win_ref[0:1, :] + x[:, 1:2] * win_ref[1:2, :]
          + x[:, 2:3] * win_ref[2:3, :]) + bin_ref[...]
    tod = jnp.clip((x[:, 1:2] * SPD).astype(jnp.int32), 0, SPD - 1)
    dow = jnp.clip((x[:, 2:3] * 7).astype(jnp.int32), 0, 6)
    oh_t = (lax.broadcasted_iota(jnp.int32, (N, SPD), 1) == tod).astype(jnp.float32)
    oh_d = (lax.broadcasted_iota(jnp.int32, (N, 7), 1) == dow).astype(jnp.float32)
    e_t = oh_t @ tod_ref[...]
    e_d = oh_d @ dow_ref[...]
    out_ref[...] = jnp.concatenate([h1, e_t, e_d], axis=-1).reshape(1, N, D)


def _embed(hist, p):
    return pl.pallas_call(
        _embed_body,
        grid=(B * T,),
        in_specs=[
            pl.BlockSpec((1, N, 3), lambda s: (s, 0, 0)),
            _full((3, 48)),
            _full((1, 48)),
            _full((SPD, 24)),
            _full((7, 24)),
        ],
        out_specs=pl.BlockSpec((1, N, D), lambda s: (s, 0, 0)),
        out_shape=jax.ShapeDtypeStruct((B * T, N, D), jnp.float32),
    )(hist.reshape(B * T, N, 3), p['W_in'], p['b_in'].reshape(1, 48),
      p['tod_emb'], p['dow_emb'])


def _shift(z, rows):
    if rows == 0:
        return z
    return jnp.concatenate(
        [jnp.zeros((rows, z.shape[1]), z.dtype), z[:-rows]], axis=0)


def _temporal_body(x_ref, emask, emaskt, wq, bq, wk, bk, wv, bv, wo, bo,
                   l1g, l1b, w1, b1, w2, b2, l2g, l2b, out_ref):
    x = x_ref[...].reshape(TN, D)
    q = (x @ wq[...] + bq[...]) * (HD ** -0.5)
    k = x @ wk[...] + bk[...]
    v = x @ wv[...] + bv[...]
    em = emask[...]
    emt = emaskt[...]
    rowt = lax.broadcasted_iota(jnp.int32, (TN, 1), 0) // N
    mx = jnp.full((TN, H), -1e9, jnp.float32)
    for o in range(T):
        s_o = (q * _shift(k, o * N)) @ em
        mx = jnp.where(rowt >= o, jnp.maximum(mx, s_o), mx)
    den = jnp.zeros((TN, H), jnp.float32)
    o_acc = jnp.zeros((TN, D), jnp.float32)
    for o in range(T):
        s_o = (q * _shift(k, o * N)) @ em
        e_o = jnp.where(rowt >= o, jnp.exp(s_o - mx), 0.0)
        den = den + e_o
        o_acc = o_acc + (e_o @ emt) * _shift(v, o * N)
    o_out = (o_acc / (den @ emt)) @ wo[...] + bo[...]
    x = _ln(x + o_out, l1g[...], l1b[...])
    f = jnp.maximum(x @ w1[...] + b1[...], 0.0) @ w2[...] + b2[...]
    out_ref[...] = _ln(x + f, l2g[...], l2b[...]).reshape(1, TN, D)


def _temporal(h, p):
    names = ['Wq', 'bq', 'Wk', 'bk', 'Wv', 'bv', 'Wo', 'bo',
             'ln1_g', 'ln1_b', 'W1', 'b1', 'W2', 'b2', 'ln2_g', 'ln2_b']
    args, specs = _params(p, names)
    emask = jnp.repeat(jnp.eye(H, dtype=jnp.float32), HD, axis=0)
    out = pl.pallas_call(
        _temporal_body,
        grid=(B,),
        in_specs=[pl.BlockSpec((1, TN, D), lambda b: (b, 0, 0)),
                  _full((D, H)), _full((H, D))] + specs,
        out_specs=pl.BlockSpec((1, TN, D), lambda b: (b, 0, 0)),
        out_shape=jax.ShapeDtypeStruct((B, TN, D), jnp.float32),
    )(h.reshape(B, TN, D), emask, emask.T, *args)
    return out.reshape(B * T, N, D)


def _gat_body(x_ref, emask, emaskt, ssrc, sdst, sdstt, wl, bl, wr, br,
              attrow, bias, l1g, l1b, w1, b1, w2, b2, l2g, l2b, out_ref):
    x = x_ref[...].reshape(N, D)
    xl = x @ wl[...] + bl[...]
    xr = x @ wr[...] + br[...]
    xs = ssrc[...] @ xl
    xd = sdst[...] @ xr
    z = xs + xd
    m = jnp.where(z > 0, z, 0.2 * z)
    logit = (m * attrow[...]) @ emask[...]
    shift = jnp.max(logit, axis=0, keepdims=True)
    ex = jnp.exp(logit - shift)
    den = sdstt[...] @ ex
    deng = sdst[...] @ den
    w = ex / jnp.maximum(deng, 1e-16)
    wx = xs * (w @ emaskt[...])
    agg = sdstt[...] @ wx + bias[...]
    x = _ln(x + agg, l1g[...], l1b[...])
    f = jnp.maximum(x @ w1[...] + b1[...], 0.0) @ w2[...] + b2[...]
    out_ref[...] = _ln(x + f, l2g[...], l2b[...]).reshape(1, N, D)


def _gat(h, ssrc, sdst, sdstt, p):
    names = ['Wl', 'bl', 'Wr', 'br', 'bias',
             'ln1_g', 'ln1_b', 'W1', 'b1', 'W2', 'b2', 'ln2_g', 'ln2_b']
    args, specs = _params(p, names)
    args.insert(4, p['att'].reshape(1, D))
    specs.insert(4, _full((1, D)))
    emask = jnp.repeat(jnp.eye(H, dtype=jnp.float32), HD, axis=0)
    return pl.pallas_call(
        _gat_body,
        grid=(B * T,),
        in_specs=[pl.BlockSpec((1, N, D), lambda s: (s, 0, 0)),
                  _full((D, H)), _full((H, D)),
                  _full(ssrc.shape), _full(sdst.shape), _full(sdstt.shape)] + specs,
        out_specs=pl.BlockSpec((1, N, D), lambda s: (s, 0, 0)),
        out_shape=jax.ShapeDtypeStruct((B * T, N, D), jnp.float32),
    )(h, emask, emask.T, ssrc, sdst, sdstt, *args)


def _head_body(x_ref, wout_ref, bout_ref, out_ref):
    x = x_ref[...].reshape(T, N, D)
    acc = x[0] @ wout_ref[0]
    for t in range(1, T):
        acc = acc + x[t] @ wout_ref[t]
    out_ref[...] = (acc + bout_ref[...]).reshape(1, N, T_OUT)


def _head(h, p):
    wout = p['W_out'].reshape(T, D, T_OUT)
    return pl.pallas_call(
        _head_body,
        grid=(B,),
        in_specs=[
            pl.BlockSpec((1, T, N, D), lambda b: (b, 0, 0, 0)),
            _full((T, D, T_OUT)),
            _full((1, T_OUT)),
        ],
        out_specs=pl.BlockSpec((1, N, T_OUT), lambda b: (b, 0, 0)),
        out_shape=jax.ShapeDtypeStruct((B, N, T_OUT), jnp.float32),
    )(h.reshape(B, T, N, D), wout, p['b_out'].reshape(1, T_OUT))


def kernel(history_data, future_data, batch_seen, epoch, train, edge_index, params):
    src, dst = edge_index[0], edge_index[1]
    ssrc = jax.nn.one_hot(src, N, dtype=jnp.float32)
    sdst = jax.nn.one_hot(dst, N, dtype=jnp.float32)
    sdstt = sdst.T

    h = _embed(history_data, params)
    for l in range(NL):
        h = _temporal(h, params['t%d' % l])
        h = _gat(h, ssrc, sdst, sdstt, params['s%d' % l])
    out = _head(h, params)
    return jnp.swapaxes(out, 1, 2)[..., None]
